# Initial kernel scaffold; baseline (speedup 1.0000x reference)
import jax
import jax.numpy as jnp
from jax.experimental import pallas as pl


def kernel(pos_ids, freqs0, freqs1, freqs2):
    raise NotImplementedError("write your pallas kernel here")


# speedup vs baseline: 2.1049x; 2.1049x over previous
import functools

import jax
import jax.numpy as jnp
from jax import lax
from jax.experimental import pallas as pl
from jax.experimental.pallas import tpu as pltpu
from jax.experimental.pallas import tpu_sc as plsc

_AXES_LENS = (1536, 512, 512)
_CW = (32, 48, 48)
_COFF = (0, 32, 80)
_OUT_W = 128

_NC = 2
_NS = 16
_NW = _NC * _NS
_CHUNK = 128


def _tables_body(a0, a1, a2, o0, o1, o2):
    for a, o in ((a0, o0), (a1, o1), (a2, o2)):
        x = a[...]
        par = lax.broadcasted_iota(jnp.int32, x.shape, 1)
        o[...] = jnp.where((par & 1) == 0, jnp.cos(x), jnp.sin(x))


def _make_ctabs(freqs0, freqs1, freqs2):
    reps = [jnp.repeat(f, 2, axis=1) for f in (freqs0, freqs1, freqs2)]
    out_shape = [
        jax.ShapeDtypeStruct((_AXES_LENS[i], _CW[i]), jnp.float32)
        for i in range(3)
    ]
    return pl.pallas_call(_tables_body, out_shape=out_shape)(*reps)


def _gather_body(ctab0, ctab1, ctab2, idx0, idx1, idx2,
                 out0, out1, out2,
                 iv0, iv1, iv2, r0, r1, r2, sem):
    wid = lax.axis_index("s") * _NC + lax.axis_index("c")
    n_chunks = idx0.shape[1]
    base = wid * (n_chunks * _CHUNK)
    pltpu.sync_copy(idx0.at[wid], iv0)
    pltpu.sync_copy(idx1.at[wid], iv1)
    pltpu.sync_copy(idx2.at[wid], iv2)
    ctabs = (ctab0, ctab1, ctab2)
    ivs = (iv0, iv1, iv2)
    rows = (r0, r1, r2)
    outs = (out0, out1, out2)
    for j in range(n_chunks):
        r0w = base + j * _CHUNK
        cps = [
            pltpu.async_copy(ctabs[a].at[ivs[a].at[j]], rows[a], sem)
            for a in range(3)
        ]
        for cp in cps:
            cp.wait()
        for a in range(3):
            pltpu.sync_copy(rows[a], outs[a].at[pl.ds(r0w, _CHUNK)])


def _sc_gather(ctab0, ctab1, ctab2, idx0, idx1, idx2, total):
    n_chunks = total // (_NW * _CHUNK)
    mesh = plsc.VectorSubcoreMesh(
        core_axis_name="c", subcore_axis_name="s",
        num_cores=_NC, num_subcores=_NS,
    )
    run = pl.kernel(
        _gather_body,
        out_type=[
            jax.ShapeDtypeStruct((total, _CW[a]), jnp.float32)
            for a in range(3)
        ],
        mesh=mesh,
        scratch_types=[
            pltpu.VMEM((n_chunks, _CHUNK), jnp.int32),
            pltpu.VMEM((n_chunks, _CHUNK), jnp.int32),
            pltpu.VMEM((n_chunks, _CHUNK), jnp.int32),
            pltpu.VMEM((_CHUNK, _CW[0]), jnp.float32),
            pltpu.VMEM((_CHUNK, _CW[1]), jnp.float32),
            pltpu.VMEM((_CHUNK, _CW[2]), jnp.float32),
            pltpu.SemaphoreType.DMA,
        ],
        compiler_params=pltpu.CompilerParams(use_tc_tiling_on_sc=False),
    )
    return run(ctab0, ctab1, ctab2, idx0, idx1, idx2)


def kernel(pos_ids, freqs0, freqs1, freqs2):
    B, N, _ = pos_ids.shape
    total = B * N
    n_chunks = total // (_NW * _CHUNK)

    ctab0, ctab1, ctab2 = _make_ctabs(freqs0, freqs1, freqs2)

    pos = pos_ids.astype(jnp.int32).reshape(total, 3)
    idxs = [
        jnp.clip(pos[:, a], 0, _AXES_LENS[a] - 1)
        .reshape(_NW, n_chunks, _CHUNK)
        for a in range(3)
    ]

    outs = _sc_gather(ctab0, ctab1, ctab2, *idxs, total)

    parts = []
    for a in range(3):
        o = outs[a].reshape(B, N, _CW[a] // 2, 2)
        parts.append(lax.complex(o[..., 0], o[..., 1]))
    return jnp.concatenate(parts, axis=-1)
